# baseline (device time: 13918 ns/iter reference)
import jax
import jax.numpy as jnp
from jax import lax
from jax.experimental import pallas as pl
from jax.experimental.pallas import tpu as pltpu

N_DEV = 8
EPS = 1e-5


def kernel(x, gamma):
    m, n_per = x.shape
    blocks = m // 128
    half = blocks // 2
    mh = m // 2

    def body(x_ref, g_ref, o_ref, acc_ref, send_sems, recv_sems):
        my = lax.axis_index("i")

        barrier_sem = pltpu.get_barrier_semaphore()
        for d in range(1, N_DEV):
            peer = lax.rem(my + d, N_DEV)
            pl.semaphore_signal(
                barrier_sem, inc=1,
                device_id=(peer,), device_id_type=pl.DeviceIdType.MESH,
            )

        def partial_sumsq(c):
            xc = x_ref[pl.ds(c * mh, mh), :]
            acc_ref[my, pl.ds(c * half, half)] = (
                jnp.sum(xc * xc, axis=1).reshape(half, 128)
            )

        def start_sends(c):
            sends = []
            for d in range(1, N_DEV):
                peer = lax.rem(my + d, N_DEV)
                rdma = pltpu.make_async_remote_copy(
                    src_ref=acc_ref.at[my, pl.ds(c * half, half)],
                    dst_ref=acc_ref.at[my, pl.ds(c * half, half)],
                    send_sem=send_sems.at[c, d],
                    recv_sem=recv_sems.at[c, my],
                    device_id=(peer,),
                    device_id_type=pl.DeviceIdType.MESH,
                )
                rdma.start()
                sends.append(rdma)
            return sends

        def wait_recvs(c):
            for d in range(1, N_DEV):
                sender = lax.rem(my + N_DEV - d, N_DEV)
                recv = pltpu.make_async_remote_copy(
                    src_ref=acc_ref.at[sender, pl.ds(c * half, half)],
                    dst_ref=acc_ref.at[sender, pl.ds(c * half, half)],
                    send_sem=send_sems.at[c, d],
                    recv_sem=recv_sems.at[c, sender],
                    device_id=(my,),
                    device_id_type=pl.DeviceIdType.MESH,
                )
                recv.wait_recv()

        eye = (
            lax.broadcasted_iota(jnp.int32, (128, 128), 0)
            == lax.broadcasted_iota(jnp.int32, (128, 128), 1)
        ).astype(jnp.float32)
        g = g_ref[...][None, :]

        def scale_cols(c):
            total = jnp.sum(acc_ref[:, pl.ds(c * half, half), :], axis=0)
            inv = lax.rsqrt(total * (1.0 / (N_DEV * n_per)) + EPS)
            return lax.dot_general(
                eye, inv,
                dimension_numbers=(((1,), (1,)), ((), ())),
                preferred_element_type=jnp.float32,
            )

        def write_out(c, cols):
            for b in range(0, half, 4):
                rs = pl.ds((c * half + b) * 128, 512)
                scale = jnp.concatenate(
                    [cols[:, b + i][:, None] for i in range(4)], axis=0
                )
                o_ref[rs, :] = (
                    x_ref[rs, :] * g * scale
                ).astype(jnp.bfloat16)

        partial_sumsq(0)
        pl.semaphore_wait(barrier_sem, N_DEV - 1)
        sends = start_sends(0)

        partial_sumsq(1)
        sends += start_sends(1)

        wait_recvs(0)
        cols_a = scale_cols(0)
        write_out(0, cols_a)

        wait_recvs(1)
        cols_b = scale_cols(1)
        write_out(1, cols_b)

        for rdma in sends:
            rdma.wait_send()

    return pl.pallas_call(
        body,
        out_shape=jax.ShapeDtypeStruct((m, n_per), jnp.bfloat16),
        in_specs=[
            pl.BlockSpec(memory_space=pltpu.VMEM),
            pl.BlockSpec(memory_space=pltpu.VMEM),
        ],
        out_specs=pl.BlockSpec(memory_space=pltpu.VMEM),
        scratch_shapes=[
            pltpu.VMEM((N_DEV, blocks, 128), jnp.float32),
            pltpu.SemaphoreType.DMA((2, N_DEV)),
            pltpu.SemaphoreType.DMA((2, N_DEV)),
        ],
        compiler_params=pltpu.CompilerParams(collective_id=0),
    )(x, gamma)
